# Initial kernel scaffold; baseline (speedup 1.0000x reference)
import jax
import jax.numpy as jnp
from jax.experimental import pallas as pl


def kernel(x, target):
    raise NotImplementedError("write your pallas kernel here")


# speedup vs baseline: 6.6247x; 6.6247x over previous
import math

import jax
import jax.numpy as jnp
from jax.experimental import pallas as pl

_SIZE = 32000
_PAD = 0
_SMOOTH = 0.1
_CONF = 1.0 - _SMOOTH
_EPS = _SMOOTH / (_SIZE - 2)
_N = 4096
_BM = 256
_BN = 6400
_C1 = _EPS * math.log(_EPS) * (_SIZE - 2) + _CONF * math.log(_CONF)


def _kl_kernel(t_ref, x_ref, o_ref):
    i = pl.program_id(0)
    j = pl.program_id(1)

    @pl.when((i == 0) & (j == 0))
    def _():
        o_ref[...] = jnp.zeros_like(o_ref)

    t = t_ref[...]
    x = x_ref[...]
    cols = jax.lax.broadcasted_iota(jnp.int32, (_BM, _BN), 1) + j * _BN
    w = jnp.where(cols == t, _CONF, _EPS)
    w = jnp.where(cols == _PAD, 0.0, w)
    w = jnp.where(t == _PAD, 0.0, w)
    acc = -jnp.sum(w * x)

    @pl.when(j == 0)
    def _():
        nonpad = jnp.sum(jnp.where(t != _PAD, 1.0, 0.0))
        o_ref[...] += (_C1 * nonpad).reshape(1, 1)

    o_ref[...] += acc.reshape(1, 1)


@jax.jit
def kernel(x, target):
    t32 = target.astype(jnp.int32).reshape(_N, 1)
    out = pl.pallas_call(
        _kl_kernel,
        grid=(_N // _BM, _SIZE // _BN),
        in_specs=[
            pl.BlockSpec((_BM, 1), lambda i, j: (i, 0)),
            pl.BlockSpec((_BM, _BN), lambda i, j: (i, j)),
        ],
        out_specs=pl.BlockSpec((1, 1), lambda i, j: (0, 0)),
        out_shape=jax.ShapeDtypeStruct((1, 1), jnp.float32),
    )(t32, x)
    return out[0, 0]


# speedup vs baseline: 7.6976x; 1.1619x over previous
import math

import jax
import jax.numpy as jnp
from jax.experimental import pallas as pl

_SIZE = 32000
_PAD = 0
_SMOOTH = 0.1
_CONF = 1.0 - _SMOOTH
_EPS = _SMOOTH / (_SIZE - 2)
_N = 4096
_BM = 256
_BN = 6400
_C1 = _EPS * math.log(_EPS) * (_SIZE - 2) + _CONF * math.log(_CONF)


def _kl_kernel(t_ref, x_ref, o_ref):
    i = pl.program_id(0)
    j = pl.program_id(1)

    @pl.when((i == 0) & (j == 0))
    def _():
        o_ref[...] = jnp.zeros_like(o_ref)

    t = t_ref[...]
    x = x_ref[...]
    cols = jax.lax.broadcasted_iota(jnp.int32, (_BM, _BN), 1) + j * _BN
    hit = jnp.sum(jnp.where(cols == t, x, 0.0), axis=1, keepdims=True)
    rowsum = jnp.sum(x, axis=1, keepdims=True)
    live = t != _PAD
    acc = jnp.sum(jnp.where(live, -_EPS, 0.0) * rowsum
                  + jnp.where(live, _EPS - _CONF, 0.0) * hit)

    @pl.when(j == 0)
    def _():
        livef = jnp.where(live, 1.0, 0.0)
        extra = jnp.sum(livef * (_EPS * x[:, 0:1] + _C1))
        o_ref[...] += extra.reshape(1, 1)

    o_ref[...] += acc.reshape(1, 1)


@jax.jit
def kernel(x, target):
    t32 = target.astype(jnp.int32).reshape(_N, 1)
    out = pl.pallas_call(
        _kl_kernel,
        grid=(_N // _BM, _SIZE // _BN),
        in_specs=[
            pl.BlockSpec((_BM, 1), lambda i, j: (i, 0)),
            pl.BlockSpec((_BM, _BN), lambda i, j: (i, j)),
        ],
        out_specs=pl.BlockSpec((1, 1), lambda i, j: (0, 0)),
        out_shape=jax.ShapeDtypeStruct((1, 1), jnp.float32),
    )(t32, x)
    return out[0, 0]


# speedup vs baseline: 7.7087x; 1.0015x over previous
import math

import jax
import jax.numpy as jnp
from jax.experimental import pallas as pl

_SIZE = 32000
_PAD = 0
_SMOOTH = 0.1
_CONF = 1.0 - _SMOOTH
_EPS = _SMOOTH / (_SIZE - 2)
_N = 4096
_BM = 256
_BN = 6400
_C1 = _EPS * math.log(_EPS) * (_SIZE - 2) + _CONF * math.log(_CONF)


def _kl_kernel(t_ref, x_ref, o_ref):
    i = pl.program_id(0)
    j = pl.program_id(1)

    @pl.when((i == 0) & (j == 0))
    def _():
        o_ref[...] = jnp.zeros_like(o_ref)

    t = t_ref[...]
    x = x_ref[...]
    cols = jax.lax.broadcasted_iota(jnp.int32, (_BM, _BN), 1) + j * _BN
    y = jnp.where(cols == t, (_CONF / _EPS) * x, x)
    rowsum = jnp.sum(y, axis=1, keepdims=True)
    live = t != _PAD
    acc = jnp.sum(jnp.where(live, -_EPS, 0.0) * rowsum)

    @pl.when(j == 0)
    def _():
        livef = jnp.where(live, 1.0, 0.0)
        extra = jnp.sum(livef * (_EPS * x[:, 0:1] + _C1))
        o_ref[...] += extra.reshape(1, 1)

    o_ref[...] += acc.reshape(1, 1)


@jax.jit
def kernel(x, target):
    t32 = target.astype(jnp.int32).reshape(_N, 1)
    out = pl.pallas_call(
        _kl_kernel,
        grid=(_N // _BM, _SIZE // _BN),
        in_specs=[
            pl.BlockSpec((_BM, 1), lambda i, j: (i, 0)),
            pl.BlockSpec((_BM, _BN), lambda i, j: (i, j)),
        ],
        out_specs=pl.BlockSpec((1, 1), lambda i, j: (0, 0)),
        out_shape=jax.ShapeDtypeStruct((1, 1), jnp.float32),
    )(t32, x)
    return out[0, 0]


# speedup vs baseline: 8.4556x; 1.0969x over previous
import math

import jax
import jax.numpy as jnp
from jax.experimental import pallas as pl

_SIZE = 32000
_PAD = 0
_SMOOTH = 0.1
_CONF = 1.0 - _SMOOTH
_EPS = _SMOOTH / (_SIZE - 2)
_N = 4096
_BM = 256
_BN = 16000
_C1 = _EPS * math.log(_EPS) * (_SIZE - 2) + _CONF * math.log(_CONF)


def _kl_kernel(t_ref, x_ref, o_ref):
    i = pl.program_id(0)
    j = pl.program_id(1)

    @pl.when((i == 0) & (j == 0))
    def _():
        o_ref[...] = jnp.zeros_like(o_ref)

    t = t_ref[...]
    x = x_ref[...]
    cols = jax.lax.broadcasted_iota(jnp.int32, (_BM, _BN), 1) + j * _BN
    y = jnp.where(cols == t, (_CONF / _EPS) * x, x)
    rowsum = jnp.sum(y, axis=1, keepdims=True)
    live = t != _PAD
    acc = jnp.sum(jnp.where(live, -_EPS, 0.0) * rowsum)

    @pl.when(j == 0)
    def _():
        livef = jnp.where(live, 1.0, 0.0)
        extra = jnp.sum(livef * (_EPS * x[:, 0:1] + _C1))
        o_ref[...] += extra.reshape(1, 1)

    o_ref[...] += acc.reshape(1, 1)


@jax.jit
def kernel(x, target):
    t32 = target.astype(jnp.int32).reshape(_N, 1)
    out = pl.pallas_call(
        _kl_kernel,
        grid=(_N // _BM, _SIZE // _BN),
        in_specs=[
            pl.BlockSpec((_BM, 1), lambda i, j: (i, 0)),
            pl.BlockSpec((_BM, _BN), lambda i, j: (i, j)),
        ],
        out_specs=pl.BlockSpec((1, 1), lambda i, j: (0, 0)),
        out_shape=jax.ShapeDtypeStruct((1, 1), jnp.float32),
    )(t32, x)
    return out[0, 0]


# speedup vs baseline: 8.4559x; 1.0000x over previous
import math

import jax
import jax.numpy as jnp
from jax.experimental import pallas as pl

_SIZE = 32000
_PAD = 0
_SMOOTH = 0.1
_CONF = 1.0 - _SMOOTH
_EPS = _SMOOTH / (_SIZE - 2)
_N = 4096
_BM = 128
_BN = 32000
_C1 = _EPS * math.log(_EPS) * (_SIZE - 2) + _CONF * math.log(_CONF)


def _kl_kernel(t_ref, x_ref, o_ref):
    i = pl.program_id(0)
    j = pl.program_id(1)

    @pl.when((i == 0) & (j == 0))
    def _():
        o_ref[...] = jnp.zeros_like(o_ref)

    t = t_ref[...]
    x = x_ref[...]
    cols = jax.lax.broadcasted_iota(jnp.int32, (_BM, _BN), 1) + j * _BN
    y = jnp.where(cols == t, (_CONF / _EPS) * x, x)
    rowsum = jnp.sum(y, axis=1, keepdims=True)
    live = t != _PAD
    acc = jnp.sum(jnp.where(live, -_EPS, 0.0) * rowsum)

    @pl.when(j == 0)
    def _():
        livef = jnp.where(live, 1.0, 0.0)
        extra = jnp.sum(livef * (_EPS * x[:, 0:1] + _C1))
        o_ref[...] += extra.reshape(1, 1)

    o_ref[...] += acc.reshape(1, 1)


@jax.jit
def kernel(x, target):
    t32 = target.astype(jnp.int32).reshape(_N, 1)
    out = pl.pallas_call(
        _kl_kernel,
        grid=(_N // _BM, _SIZE // _BN),
        in_specs=[
            pl.BlockSpec((_BM, 1), lambda i, j: (i, 0)),
            pl.BlockSpec((_BM, _BN), lambda i, j: (i, j)),
        ],
        out_specs=pl.BlockSpec((1, 1), lambda i, j: (0, 0)),
        out_shape=jax.ShapeDtypeStruct((1, 1), jnp.float32),
    )(t32, x)
    return out[0, 0]
